# Initial kernel scaffold; baseline (speedup 1.0000x reference)
import jax
import jax.numpy as jnp
from jax.experimental import pallas as pl


def kernel(input_ids, emb0, emb1, emb2):
    raise NotImplementedError("write your pallas kernel here")


# speedup vs baseline: 14.4248x; 14.4248x over previous
import functools

import jax
import jax.numpy as jnp
from jax import lax
from jax.experimental import pallas as pl
from jax.experimental.pallas import tpu as pltpu
from jax.experimental.pallas import tpu_sc as plsc

B = 4
S = 2048
D = 1024
MD0, MD1, MD2 = 2050, 64, 64
OFFSET = 2

NC = 2
NS = 16
NW = NC * NS
ROWS = B * S
ROWS_PER_W = ROWS // NW
CHUNK = 32
NCHUNK = ROWS_PER_W // CHUNK


def _shift_right(x, k, fill):
    pad = jnp.full((B, k), fill, dtype=x.dtype)
    return jnp.concatenate([pad, x[:, : S - k]], axis=1)


def _cumsum(x):
    k = 1
    while k < S:
        x = x + _shift_right(x, k, 0)
        k *= 2
    return x


def _cummax(x, fill):
    k = 1
    while k < S:
        x = jnp.maximum(x, _shift_right(x, k, fill))
        k *= 2
    return x


def _maps_body(ids_ref, m0_ref, m1_ref, m2_ref, cnt_ref):
    tok = ids_ref[...]
    c1 = jnp.logical_and(tok >= 5, tok <= 8)
    c2 = jnp.logical_and(tok >= 9, tok <= 10)
    i32 = jnp.int32
    t = lax.broadcasted_iota(i32, (B, S), 1)
    e = (tok == 1).astype(i32)
    done = _cumsum(e) > 0
    s1 = _cumsum(c1.astype(i32))
    cc2 = _cumsum(c2.astype(i32))
    lastreset = _cummax(jnp.where(jnp.logical_or(c1, c2), t, -1), -1)
    n0raw = jnp.where(lastreset >= 0, t - lastreset, t + 1 + OFFSET)
    ov0 = n0raw == MD0
    n0 = jnp.where(ov0, 0, n0raw)
    v = _cummax(jnp.where(c2, s1, 0), 0)
    n1c = s1 - v
    wrap1 = jnp.logical_and(c1, (n1c & 63) == 0)
    w = _cumsum(wrap1.astype(i32))
    n1 = (n1c & 63) + ov0.astype(i32)
    n2 = (cc2 + w) & 63
    m0_ref[...] = jnp.where(done, MD0 - 1, n0)
    m1_ref[...] = jnp.where(done, MD1 - 1, n1)
    m2_ref[...] = jnp.where(done, MD2 - 1, n2)
    p = jnp.sum(jnp.logical_not(done).astype(i32), axis=1, keepdims=True)
    sel = t == (p - 1)
    f0 = jnp.sum(jnp.where(sel, n0, 0), axis=1, keepdims=True)
    f1 = jnp.sum(jnp.where(sel, n1, 0), axis=1, keepdims=True)
    f2 = jnp.sum(jnp.where(sel, n2, 0), axis=1, keepdims=True)
    f0 = jnp.where(p == 0, OFFSET, f0)
    f1 = jnp.where(p == 0, 0, f1)
    f2 = jnp.where(p == 0, 0, f2)
    col = lax.broadcasted_iota(i32, (B, 128), 1)
    cnt = jnp.where(col == 0, f0, jnp.where(col == 1, f1, jnp.where(col == 2, f2, 0)))
    cnt_ref[...] = cnt


def _compute_maps(input_ids, interpret=False):
    out = pl.pallas_call(
        _maps_body,
        out_shape=[
            jax.ShapeDtypeStruct((B, S), jnp.int32),
            jax.ShapeDtypeStruct((B, S), jnp.int32),
            jax.ShapeDtypeStruct((B, S), jnp.int32),
            jax.ShapeDtypeStruct((B, 128), jnp.int32),
        ],
        interpret=interpret,
    )(input_ids)
    return out


def _gather_body(m0h, m1h, m2h, e0h, e1h, e2h, outh,
                 i0, i1, i2, b0, b1, b2, s0, s1, s2):
    wid = lax.axis_index("s") * NC + lax.axis_index("c")
    base = wid * ROWS_PER_W

    def chunk_body(ci, carry):
        r0 = base + ci * CHUNK
        pltpu.sync_copy(m0h.at[pl.ds(r0, CHUNK)], i0)
        pltpu.sync_copy(m1h.at[pl.ds(r0, CHUNK)], i1)
        pltpu.sync_copy(m2h.at[pl.ds(r0, CHUNK)], i2)
        c0 = pltpu.async_copy(e0h.at[i0], b0, s0)
        c1 = pltpu.async_copy(e1h.at[i1], b1, s1)
        c2 = pltpu.async_copy(e2h.at[i2], b2, s2)
        c0.wait()
        c1.wait()
        c2.wait()

        def add_row(r, carry2):
            for cb in range(D // 16):
                sl = pl.ds(cb * 16, 16)
                v = b1[r, sl] + b2[r, sl]
                plsc.addupdate(b0.at[r, sl], v)
            return carry2

        lax.fori_loop(0, CHUNK, add_row, 0, unroll=False)
        pltpu.sync_copy(b0, outh.at[pl.ds(r0, CHUNK)])
        return carry

    lax.fori_loop(0, NCHUNK, chunk_body, 0, unroll=False)


def _gather_sum(m0f, m1f, m2f, emb0, emb1, emb2):
    mesh = plsc.VectorSubcoreMesh(
        core_axis_name="c", subcore_axis_name="s",
        num_cores=NC, num_subcores=NS)
    kern = pl.kernel(
        _gather_body,
        out_type=jax.ShapeDtypeStruct((ROWS, D), jnp.float32),
        mesh=mesh,
        scratch_types=[
            pltpu.VMEM((CHUNK,), jnp.int32),
            pltpu.VMEM((CHUNK,), jnp.int32),
            pltpu.VMEM((CHUNK,), jnp.int32),
            pltpu.VMEM((CHUNK, D), jnp.float32),
            pltpu.VMEM((CHUNK, D), jnp.float32),
            pltpu.VMEM((CHUNK, D), jnp.float32),
            pltpu.SemaphoreType.DMA,
            pltpu.SemaphoreType.DMA,
            pltpu.SemaphoreType.DMA,
        ],
    )
    return kern(m0f, m1f, m2f, emb0, emb1, emb2)


@jax.jit
def kernel(input_ids, emb0, emb1, emb2):
    m0, m1, m2, cnt = _compute_maps(input_ids)
    counters = cnt[:, :3]
    out = _gather_sum(m0.reshape(ROWS), m1.reshape(ROWS), m2.reshape(ROWS),
                      emb0, emb1, emb2)
    return out.reshape(B, S, D), counters


# speedup vs baseline: 24.8018x; 1.7194x over previous
import functools

import jax
import jax.numpy as jnp
from jax import lax
from jax.experimental import pallas as pl
from jax.experimental.pallas import tpu as pltpu
from jax.experimental.pallas import tpu_sc as plsc

B = 4
S = 2048
D = 1024
H = D // 2
MD0, MD1, MD2 = 2050, 64, 64
OFFSET = 2

NC = 2
NS = 16
ROWS = B * S
ROWS_PER_T = ROWS // NS
CHUNK = 32
NCHUNK = ROWS_PER_T // CHUNK


def _shift_right(x, k, fill):
    pad = jnp.full((B, k), fill, dtype=x.dtype)
    return jnp.concatenate([pad, x[:, : S - k]], axis=1)


def _cumsum(x):
    k = 1
    while k < S:
        x = x + _shift_right(x, k, 0)
        k *= 2
    return x


def _cummax(x, fill):
    k = 1
    while k < S:
        x = jnp.maximum(x, _shift_right(x, k, fill))
        k *= 2
    return x


def _maps_body(ids_ref, m0_ref, m1_ref, m2_ref, cnt_ref):
    tok = ids_ref[...]
    c1 = jnp.logical_and(tok >= 5, tok <= 8)
    c2 = jnp.logical_and(tok >= 9, tok <= 10)
    i32 = jnp.int32
    t = lax.broadcasted_iota(i32, (B, S), 1)
    e = (tok == 1).astype(i32)
    done = _cumsum(e) > 0
    s1 = _cumsum(c1.astype(i32))
    cc2 = _cumsum(c2.astype(i32))
    lastreset = _cummax(jnp.where(jnp.logical_or(c1, c2), t, -1), -1)
    n0raw = jnp.where(lastreset >= 0, t - lastreset, t + 1 + OFFSET)
    ov0 = n0raw == MD0
    n0 = jnp.where(ov0, 0, n0raw)
    v = _cummax(jnp.where(c2, s1, 0), 0)
    n1c = s1 - v
    wrap1 = jnp.logical_and(c1, (n1c & 63) == 0)
    w = _cumsum(wrap1.astype(i32))
    n1 = (n1c & 63) + ov0.astype(i32)
    n2 = (cc2 + w) & 63
    m0_ref[...] = jnp.where(done, MD0 - 1, n0)
    m1_ref[...] = jnp.where(done, MD1 - 1, n1)
    m2_ref[...] = jnp.where(done, MD2 - 1, n2)
    p = jnp.sum(jnp.logical_not(done).astype(i32), axis=1, keepdims=True)
    sel = t == (p - 1)
    f0 = jnp.sum(jnp.where(sel, n0, 0), axis=1, keepdims=True)
    f1 = jnp.sum(jnp.where(sel, n1, 0), axis=1, keepdims=True)
    f2 = jnp.sum(jnp.where(sel, n2, 0), axis=1, keepdims=True)
    f0 = jnp.where(p == 0, OFFSET, f0)
    f1 = jnp.where(p == 0, 0, f1)
    f2 = jnp.where(p == 0, 0, f2)
    col = lax.broadcasted_iota(i32, (B, 128), 1)
    cnt = jnp.where(col == 0, f0, jnp.where(col == 1, f1, jnp.where(col == 2, f2, 0)))
    cnt_ref[...] = cnt


def _compute_maps(input_ids, interpret=False):
    out = pl.pallas_call(
        _maps_body,
        out_shape=[
            jax.ShapeDtypeStruct((B, S), jnp.int32),
            jax.ShapeDtypeStruct((B, S), jnp.int32),
            jax.ShapeDtypeStruct((B, S), jnp.int32),
            jax.ShapeDtypeStruct((B, 128), jnp.int32),
        ],
        interpret=interpret,
    )(input_ids)
    return out


def _gather_body(m0h, m1h, m2h, e0h, e1h, e2h, outh,
                 i0, i1, i2, b0, loc1, loc2, s0):
    c = lax.axis_index("c")
    s = lax.axis_index("s")
    base = s * ROWS_PER_T

    pltpu.sync_copy(e1h.at[pl.ds(c * MD1, MD1)], loc1)
    pltpu.sync_copy(e2h.at[pl.ds(c * MD2, MD2)], loc2)

    def chunk_body(ci, carry):
        r0 = base + ci * CHUNK
        pltpu.sync_copy(m0h.at[pl.ds(r0, CHUNK)], i0)
        pltpu.sync_copy(m1h.at[pl.ds(r0, CHUNK)], i1)
        pltpu.sync_copy(m2h.at[pl.ds(r0, CHUNK)], i2)
        for h in range(CHUNK // 16):
            sl = pl.ds(h * 16, 16)
            i0.at[sl][...] = i0.at[sl][...] + c * MD0
        pltpu.async_copy(e0h.at[i0], b0, s0).wait()

        for h in range(CHUNK // 16):
            sl = pl.ds(h * 16, 16)
            i1v = i1.at[sl][...]
            i2v = i2.at[sl][...]
            rowv = h * 16 + lax.iota(jnp.int32, 16)

            def col_body(cc, carry2):
                cvec = jnp.full((16,), cc, jnp.int32)
                v = (plsc.load_gather(loc1, [i1v, cvec])
                     + plsc.load_gather(loc2, [i2v, cvec]))
                plsc.addupdate_scatter(b0, [rowv, cvec], v)
                return carry2

            lax.fori_loop(0, H, col_body, 0, unroll=4)

        pltpu.sync_copy(b0, outh.at[pl.ds(r0, CHUNK), c])
        return carry

    lax.fori_loop(0, NCHUNK, chunk_body, 0, unroll=False)


def _gather_sum(m0f, m1f, m2f, e0s, e1s, e2s):
    mesh = plsc.VectorSubcoreMesh(
        core_axis_name="c", subcore_axis_name="s",
        num_cores=NC, num_subcores=NS)
    kern = pl.kernel(
        _gather_body,
        out_type=jax.ShapeDtypeStruct((ROWS, NC, H), jnp.float32),
        mesh=mesh,
        compiler_params=pltpu.CompilerParams(needs_layout_passes=False),
        scratch_types=[
            pltpu.VMEM((CHUNK,), jnp.int32),
            pltpu.VMEM((CHUNK,), jnp.int32),
            pltpu.VMEM((CHUNK,), jnp.int32),
            pltpu.VMEM((CHUNK, H), jnp.float32),
            pltpu.VMEM((MD1, H), jnp.float32),
            pltpu.VMEM((MD2, H), jnp.float32),
            pltpu.SemaphoreType.DMA,
        ],
    )
    return kern(m0f, m1f, m2f, e0s, e1s, e2s)


@jax.jit
def kernel(input_ids, emb0, emb1, emb2):
    m0, m1, m2, cnt = _compute_maps(input_ids)
    counters = cnt[:, :3]
    e0s = jnp.concatenate([emb0[:, :H], emb0[:, H:]], axis=0)
    e1s = jnp.concatenate([emb1[:, :H], emb1[:, H:]], axis=0)
    e2s = jnp.concatenate([emb2[:, :H], emb2[:, H:]], axis=0)
    out = _gather_sum(m0.reshape(ROWS), m1.reshape(ROWS), m2.reshape(ROWS),
                      e0s, e1s, e2s)
    return out.reshape(B, S, D), counters


# speedup vs baseline: 29.6748x; 1.1965x over previous
import functools

import jax
import jax.numpy as jnp
from jax import lax
from jax.experimental import pallas as pl
from jax.experimental.pallas import tpu as pltpu
from jax.experimental.pallas import tpu_sc as plsc

B = 4
S = 2048
D = 1024
H = D // 2
MD0, MD1, MD2 = 2050, 64, 64
OFFSET = 2

NC = 2
NS = 16
ROWS = B * S
ROWS_PER_T = ROWS // NS
CHUNK = 32
NCHUNK = ROWS_PER_T // CHUNK


def _shift_right(x, k, fill):
    pad = jnp.full((B, k), fill, dtype=x.dtype)
    return jnp.concatenate([pad, x[:, : S - k]], axis=1)


def _cumsum(x):
    k = 1
    while k < S:
        x = x + _shift_right(x, k, 0)
        k *= 2
    return x


def _cummax(x, fill):
    k = 1
    while k < S:
        x = jnp.maximum(x, _shift_right(x, k, fill))
        k *= 2
    return x


def _maps_body(ids_ref, m0_ref, m1_ref, m2_ref, cnt_ref):
    tok = ids_ref[...]
    c1 = jnp.logical_and(tok >= 5, tok <= 8)
    c2 = jnp.logical_and(tok >= 9, tok <= 10)
    i32 = jnp.int32
    t = lax.broadcasted_iota(i32, (B, S), 1)
    e = (tok == 1).astype(i32)
    done = _cumsum(e) > 0
    s1 = _cumsum(c1.astype(i32))
    cc2 = _cumsum(c2.astype(i32))
    lastreset = _cummax(jnp.where(jnp.logical_or(c1, c2), t, -1), -1)
    n0raw = jnp.where(lastreset >= 0, t - lastreset, t + 1 + OFFSET)
    ov0 = n0raw == MD0
    n0 = jnp.where(ov0, 0, n0raw)
    v = _cummax(jnp.where(c2, s1, 0), 0)
    n1c = s1 - v
    wrap1 = jnp.logical_and(c1, (n1c & 63) == 0)
    w = _cumsum(wrap1.astype(i32))
    n1 = (n1c & 63) + ov0.astype(i32)
    n2 = (cc2 + w) & 63
    m0_ref[...] = jnp.where(done, MD0 - 1, n0)
    m1_ref[...] = jnp.where(done, MD1 - 1, n1)
    m2_ref[...] = jnp.where(done, MD2 - 1, n2)
    p = jnp.sum(jnp.logical_not(done).astype(i32), axis=1, keepdims=True)
    sel = t == (p - 1)
    f0 = jnp.sum(jnp.where(sel, n0, 0), axis=1, keepdims=True)
    f1 = jnp.sum(jnp.where(sel, n1, 0), axis=1, keepdims=True)
    f2 = jnp.sum(jnp.where(sel, n2, 0), axis=1, keepdims=True)
    f0 = jnp.where(p == 0, OFFSET, f0)
    f1 = jnp.where(p == 0, 0, f1)
    f2 = jnp.where(p == 0, 0, f2)
    col = lax.broadcasted_iota(i32, (B, 128), 1)
    cnt = jnp.where(col == 0, f0, jnp.where(col == 1, f1, jnp.where(col == 2, f2, 0)))
    cnt_ref[...] = cnt


def _compute_maps(input_ids, interpret=False):
    out = pl.pallas_call(
        _maps_body,
        out_shape=[
            jax.ShapeDtypeStruct((B, S), jnp.int32),
            jax.ShapeDtypeStruct((B, S), jnp.int32),
            jax.ShapeDtypeStruct((B, S), jnp.int32),
            jax.ShapeDtypeStruct((B, 128), jnp.int32),
        ],
        interpret=interpret,
    )(input_ids)
    return out


def _gather_body(m0h, m1h, m2h, e0h, e1h, e2h, outh,
                 i0, i1, i2, b0, loc1, loc2, s0):
    c = lax.axis_index("c")
    s = lax.axis_index("s")
    base = s * ROWS_PER_T

    pltpu.sync_copy(e1h.at[pl.ds(c * MD1, MD1)], loc1)
    pltpu.sync_copy(e2h.at[pl.ds(c * MD2, MD2)], loc2)

    def chunk_body(ci, carry):
        r0 = base + ci * CHUNK
        pltpu.sync_copy(m0h.at[pl.ds(r0, CHUNK)], i0)
        pltpu.sync_copy(m1h.at[pl.ds(r0, CHUNK)], i1)
        pltpu.sync_copy(m2h.at[pl.ds(r0, CHUNK)], i2)
        for h in range(CHUNK // 16):
            sl = pl.ds(h * 16, 16)
            i0.at[sl][...] = i0.at[sl][...] + c * MD0
        pltpu.async_copy(e0h.at[i0], b0, s0).wait()

        for h in range(CHUNK // 16):
            sl = pl.ds(h * 16, 16)
            i1v = i1.at[sl][...]
            i2v = i2.at[sl][...]
            for r in range(16):
                m1r = i1v[r]
                m2r = i2v[r]
                row = h * 16 + r

                def col_body(cb, carry2, m1r=m1r, m2r=m2r, row=row):
                    csl = pl.ds(cb * 16, 16)
                    v = loc1.at[m1r, csl][...] + loc2.at[m2r, csl][...]
                    plsc.addupdate(b0.at[row, csl], v)
                    return carry2

                lax.fori_loop(0, H // 16, col_body, 0, unroll=4)

        pltpu.sync_copy(b0, outh.at[pl.ds(r0, CHUNK), c])
        return carry

    lax.fori_loop(0, NCHUNK, chunk_body, 0, unroll=False)


def _gather_sum(m0f, m1f, m2f, e0s, e1s, e2s):
    mesh = plsc.VectorSubcoreMesh(
        core_axis_name="c", subcore_axis_name="s",
        num_cores=NC, num_subcores=NS)
    kern = pl.kernel(
        _gather_body,
        out_type=jax.ShapeDtypeStruct((ROWS, NC, H), jnp.float32),
        mesh=mesh,
        compiler_params=pltpu.CompilerParams(needs_layout_passes=False),
        scratch_types=[
            pltpu.VMEM((CHUNK,), jnp.int32),
            pltpu.VMEM((CHUNK,), jnp.int32),
            pltpu.VMEM((CHUNK,), jnp.int32),
            pltpu.VMEM((CHUNK, H), jnp.float32),
            pltpu.VMEM((MD1, H), jnp.float32),
            pltpu.VMEM((MD2, H), jnp.float32),
            pltpu.SemaphoreType.DMA,
        ],
    )
    return kern(m0f, m1f, m2f, e0s, e1s, e2s)


@jax.jit
def kernel(input_ids, emb0, emb1, emb2):
    m0, m1, m2, cnt = _compute_maps(input_ids)
    counters = cnt[:, :3]
    e0s = jnp.concatenate([emb0[:, :H], emb0[:, H:]], axis=0)
    e1s = jnp.concatenate([emb1[:, :H], emb1[:, H:]], axis=0)
    e2s = jnp.concatenate([emb2[:, :H], emb2[:, H:]], axis=0)
    out = _gather_sum(m0.reshape(ROWS), m1.reshape(ROWS), m2.reshape(ROWS),
                      e0s, e1s, e2s)
    return out.reshape(B, S, D), counters


# speedup vs baseline: 49.6498x; 1.6731x over previous
import functools

import jax
import jax.numpy as jnp
from jax import lax
from jax.experimental import pallas as pl
from jax.experimental.pallas import tpu as pltpu
from jax.experimental.pallas import tpu_sc as plsc

B = 4
S = 2048
D = 1024
H = D // 2
MD0, MD1, MD2 = 2050, 64, 64
OFFSET = 2

NC = 2
NS = 16
ROWS = B * S
ROWS_PER_T = ROWS // NS
CHUNK = 32
NCHUNK = ROWS_PER_T // CHUNK


def _shift_right(x, k, fill):
    pad = jnp.full((B, k), fill, dtype=x.dtype)
    return jnp.concatenate([pad, x[:, : S - k]], axis=1)


def _cumsum(x):
    k = 1
    while k < S:
        x = x + _shift_right(x, k, 0)
        k *= 2
    return x


def _cummax(x, fill):
    k = 1
    while k < S:
        x = jnp.maximum(x, _shift_right(x, k, fill))
        k *= 2
    return x


def _maps_body(ids_ref, m0_ref, m1_ref, m2_ref, cnt_ref):
    tok = ids_ref[...]
    c1 = jnp.logical_and(tok >= 5, tok <= 8)
    c2 = jnp.logical_and(tok >= 9, tok <= 10)
    i32 = jnp.int32
    t = lax.broadcasted_iota(i32, (B, S), 1)
    e = (tok == 1).astype(i32)
    done = _cumsum(e) > 0
    s1 = _cumsum(c1.astype(i32))
    cc2 = _cumsum(c2.astype(i32))
    lastreset = _cummax(jnp.where(jnp.logical_or(c1, c2), t, -1), -1)
    n0raw = jnp.where(lastreset >= 0, t - lastreset, t + 1 + OFFSET)
    ov0 = n0raw == MD0
    n0 = jnp.where(ov0, 0, n0raw)
    v = _cummax(jnp.where(c2, s1, 0), 0)
    n1c = s1 - v
    wrap1 = jnp.logical_and(c1, (n1c & 63) == 0)
    w = _cumsum(wrap1.astype(i32))
    n1 = (n1c & 63) + ov0.astype(i32)
    n2 = (cc2 + w) & 63
    m0_ref[...] = jnp.where(done, MD0 - 1, n0)
    m1_ref[...] = jnp.where(done, MD1 - 1, n1)
    m2_ref[...] = jnp.where(done, MD2 - 1, n2)
    p = jnp.sum(jnp.logical_not(done).astype(i32), axis=1, keepdims=True)
    sel = t == (p - 1)
    f0 = jnp.sum(jnp.where(sel, n0, 0), axis=1, keepdims=True)
    f1 = jnp.sum(jnp.where(sel, n1, 0), axis=1, keepdims=True)
    f2 = jnp.sum(jnp.where(sel, n2, 0), axis=1, keepdims=True)
    f0 = jnp.where(p == 0, OFFSET, f0)
    f1 = jnp.where(p == 0, 0, f1)
    f2 = jnp.where(p == 0, 0, f2)
    col = lax.broadcasted_iota(i32, (B, 128), 1)
    cnt = jnp.where(col == 0, f0, jnp.where(col == 1, f1, jnp.where(col == 2, f2, 0)))
    cnt_ref[...] = cnt


def _compute_maps(input_ids, interpret=False):
    out = pl.pallas_call(
        _maps_body,
        out_shape=[
            jax.ShapeDtypeStruct((B, S), jnp.int32),
            jax.ShapeDtypeStruct((B, S), jnp.int32),
            jax.ShapeDtypeStruct((B, S), jnp.int32),
            jax.ShapeDtypeStruct((B, 128), jnp.int32),
        ],
        interpret=interpret,
    )(input_ids)
    return out


def _gather_body(m0h, m1h, m2h, e0h, e1h, e2h, outh,
                 i0, i1, i2, b0, loc1, loc2, s0):
    c = lax.axis_index("c")
    s = lax.axis_index("s")
    base = s * ROWS_PER_T

    pltpu.sync_copy(e1h.at[pl.ds(c * MD1, MD1)], loc1)
    pltpu.sync_copy(e2h.at[pl.ds(c * MD2, MD2)], loc2)

    def chunk_body(ci, carry):
        r0 = base + ci * CHUNK
        pltpu.sync_copy(m0h.at[pl.ds(r0, CHUNK)], i0)
        pltpu.sync_copy(m1h.at[pl.ds(r0, CHUNK)], i1)
        pltpu.sync_copy(m2h.at[pl.ds(r0, CHUNK)], i2)
        for h in range(CHUNK // 16):
            sl = pl.ds(h * 16, 16)
            i0.at[sl][...] = i0.at[sl][...] + c * MD0
        pltpu.async_copy(e0h.at[i0], b0, s0).wait()

        for h in range(CHUNK // 16):
            sl = pl.ds(h * 16, 16)
            i1v = i1.at[sl][...]
            i2v = i2.at[sl][...]
            for r in range(16):
                m1r = i1v[r]
                m2r = i2v[r]
                row = h * 16 + r

                @plsc.parallel_loop(0, H // 16, 1, unroll=4)
                def col_body(cb, m1r=m1r, m2r=m2r, row=row):
                    csl = pl.ds(cb * 16, 16)
                    v = loc1.at[m1r, csl][...] + loc2.at[m2r, csl][...]
                    plsc.addupdate(b0.at[row, csl], v)

        pltpu.sync_copy(b0, outh.at[pl.ds(r0, CHUNK), c])
        return carry

    lax.fori_loop(0, NCHUNK, chunk_body, 0, unroll=False)


def _gather_sum(m0f, m1f, m2f, e0s, e1s, e2s):
    mesh = plsc.VectorSubcoreMesh(
        core_axis_name="c", subcore_axis_name="s",
        num_cores=NC, num_subcores=NS)
    kern = pl.kernel(
        _gather_body,
        out_type=jax.ShapeDtypeStruct((ROWS, NC, H), jnp.float32),
        mesh=mesh,
        compiler_params=pltpu.CompilerParams(needs_layout_passes=False),
        scratch_types=[
            pltpu.VMEM((CHUNK,), jnp.int32),
            pltpu.VMEM((CHUNK,), jnp.int32),
            pltpu.VMEM((CHUNK,), jnp.int32),
            pltpu.VMEM((CHUNK, H), jnp.float32),
            pltpu.VMEM((MD1, H), jnp.float32),
            pltpu.VMEM((MD2, H), jnp.float32),
            pltpu.SemaphoreType.DMA,
        ],
    )
    return kern(m0f, m1f, m2f, e0s, e1s, e2s)


@jax.jit
def kernel(input_ids, emb0, emb1, emb2):
    m0, m1, m2, cnt = _compute_maps(input_ids)
    counters = cnt[:, :3]
    e0s = jnp.concatenate([emb0[:, :H], emb0[:, H:]], axis=0)
    e1s = jnp.concatenate([emb1[:, :H], emb1[:, H:]], axis=0)
    e2s = jnp.concatenate([emb2[:, :H], emb2[:, H:]], axis=0)
    out = _gather_sum(m0.reshape(ROWS), m1.reshape(ROWS), m2.reshape(ROWS),
                      e0s, e1s, e2s)
    return out.reshape(B, S, D), counters


# speedup vs baseline: 61.2728x; 1.2341x over previous
import functools

import jax
import jax.numpy as jnp
from jax import lax
from jax.experimental import pallas as pl
from jax.experimental.pallas import tpu as pltpu
from jax.experimental.pallas import tpu_sc as plsc

B = 4
S = 2048
D = 1024
H = D // 2
MD0, MD1, MD2 = 2050, 64, 64
OFFSET = 2

NC = 2
NS = 16
ROWS = B * S
ROWS_PER_T = ROWS // NS
CHUNK = 32
NCHUNK = ROWS_PER_T // CHUNK


def _shift_right(x, k, fill):
    pad = jnp.full((B, k), fill, dtype=x.dtype)
    return jnp.concatenate([pad, x[:, : S - k]], axis=1)


def _cumsum(x):
    k = 1
    while k < S:
        x = x + _shift_right(x, k, 0)
        k *= 2
    return x


def _cummax(x, fill):
    k = 1
    while k < S:
        x = jnp.maximum(x, _shift_right(x, k, fill))
        k *= 2
    return x


def _maps_body(ids_ref, m0_ref, m1_ref, m2_ref, cnt_ref):
    tok = ids_ref[...]
    c1 = jnp.logical_and(tok >= 5, tok <= 8)
    c2 = jnp.logical_and(tok >= 9, tok <= 10)
    i32 = jnp.int32
    t = lax.broadcasted_iota(i32, (B, S), 1)
    e = (tok == 1).astype(i32)
    done = _cumsum(e) > 0
    s1 = _cumsum(c1.astype(i32))
    cc2 = _cumsum(c2.astype(i32))
    lastreset = _cummax(jnp.where(jnp.logical_or(c1, c2), t, -1), -1)
    n0raw = jnp.where(lastreset >= 0, t - lastreset, t + 1 + OFFSET)
    ov0 = n0raw == MD0
    n0 = jnp.where(ov0, 0, n0raw)
    v = _cummax(jnp.where(c2, s1, 0), 0)
    n1c = s1 - v
    wrap1 = jnp.logical_and(c1, (n1c & 63) == 0)
    w = _cumsum(wrap1.astype(i32))
    n1 = (n1c & 63) + ov0.astype(i32)
    n2 = (cc2 + w) & 63
    m0_ref[...] = jnp.where(done, MD0 - 1, n0)
    m1_ref[...] = jnp.where(done, MD1 - 1, n1)
    m2_ref[...] = jnp.where(done, MD2 - 1, n2)
    p = jnp.sum(jnp.logical_not(done).astype(i32), axis=1, keepdims=True)
    sel = t == (p - 1)
    f0 = jnp.sum(jnp.where(sel, n0, 0), axis=1, keepdims=True)
    f1 = jnp.sum(jnp.where(sel, n1, 0), axis=1, keepdims=True)
    f2 = jnp.sum(jnp.where(sel, n2, 0), axis=1, keepdims=True)
    f0 = jnp.where(p == 0, OFFSET, f0)
    f1 = jnp.where(p == 0, 0, f1)
    f2 = jnp.where(p == 0, 0, f2)
    col = lax.broadcasted_iota(i32, (B, 128), 1)
    cnt = jnp.where(col == 0, f0, jnp.where(col == 1, f1, jnp.where(col == 2, f2, 0)))
    cnt_ref[...] = cnt


def _compute_maps(input_ids, interpret=False):
    out = pl.pallas_call(
        _maps_body,
        out_shape=[
            jax.ShapeDtypeStruct((B, S), jnp.int32),
            jax.ShapeDtypeStruct((B, S), jnp.int32),
            jax.ShapeDtypeStruct((B, S), jnp.int32),
            jax.ShapeDtypeStruct((B, 128), jnp.int32),
        ],
        interpret=interpret,
    )(input_ids)
    return out


def _gather_body(m0h, m1h, m2h, e0h, e1h, e2h, outh,
                 i0, i1, i2, ba, bb, loc1, loc2, sa, sb, si):
    c = lax.axis_index("c")
    s = lax.axis_index("s")
    base = s * ROWS_PER_T

    ci0 = pltpu.async_copy(m0h.at[pl.ds(base, ROWS_PER_T)], i0, si)
    pltpu.sync_copy(e1h.at[pl.ds(c * MD1, MD1)], loc1)
    pltpu.sync_copy(e2h.at[pl.ds(c * MD2, MD2)], loc2)
    ci0.wait()
    ci1 = pltpu.async_copy(m1h.at[pl.ds(base, ROWS_PER_T)], i1, si)
    ci2 = pltpu.async_copy(m2h.at[pl.ds(base, ROWS_PER_T)], i2, si)
    off = c * MD0

    @plsc.parallel_loop(0, ROWS_PER_T // 16, 1, unroll=4)
    def rebase(h):
        sl = pl.ds(h * 16, 16)
        i0.at[sl][...] = i0.at[sl][...] + off

    ci1.wait()
    ci2.wait()

    def do_adds(buf, ci):
        for h in range(CHUNK // 16):
            sl = pl.ds(ci * CHUNK + h * 16, 16)
            i1v = i1.at[sl][...]
            i2v = i2.at[sl][...]
            for r in range(16):
                m1r = i1v[r]
                m2r = i2v[r]
                row = h * 16 + r

                @plsc.parallel_loop(0, H // 16, 1, unroll=4)
                def col_body(cb, m1r=m1r, m2r=m2r, row=row):
                    csl = pl.ds(cb * 16, 16)
                    v = loc1.at[m1r, csl][...] + loc2.at[m2r, csl][...]
                    plsc.addupdate(buf.at[row, csl], v)

    def gather_start(buf, sem, ci):
        return pltpu.async_copy(e0h.at[i0.at[pl.ds(ci * CHUNK, CHUNK)]],
                                buf, sem)

    gather_start(ba, sa, 0)

    def step_pair(gi, carry):
        ci = gi * 2

        def one(buf, sem, obuf, osem, ci):
            nxt = ci + 1

            @pl.when(nxt < NCHUNK)
            def _():
                gather_start(obuf, osem, nxt)

            pltpu.make_async_copy(e0h.at[i0.at[pl.ds(ci * CHUNK, CHUNK)]],
                                  buf, sem).wait()
            do_adds(buf, ci)
            pltpu.sync_copy(buf, outh.at[pl.ds(base + ci * CHUNK, CHUNK), c])

        one(ba, sa, bb, sb, ci)
        one(bb, sb, ba, sa, ci + 1)
        return carry

    lax.fori_loop(0, NCHUNK // 2, step_pair, 0, unroll=False)


def _gather_sum(m0f, m1f, m2f, e0s, e1s, e2s):
    mesh = plsc.VectorSubcoreMesh(
        core_axis_name="c", subcore_axis_name="s",
        num_cores=NC, num_subcores=NS)
    kern = pl.kernel(
        _gather_body,
        out_type=jax.ShapeDtypeStruct((ROWS, NC, H), jnp.float32),
        mesh=mesh,
        compiler_params=pltpu.CompilerParams(needs_layout_passes=False),
        scratch_types=[
            pltpu.VMEM((ROWS_PER_T,), jnp.int32),
            pltpu.VMEM((ROWS_PER_T,), jnp.int32),
            pltpu.VMEM((ROWS_PER_T,), jnp.int32),
            pltpu.VMEM((CHUNK, H), jnp.float32),
            pltpu.VMEM((CHUNK, H), jnp.float32),
            pltpu.VMEM((MD1, H), jnp.float32),
            pltpu.VMEM((MD2, H), jnp.float32),
            pltpu.SemaphoreType.DMA,
            pltpu.SemaphoreType.DMA,
            pltpu.SemaphoreType.DMA,
        ],
    )
    return kern(m0f, m1f, m2f, e0s, e1s, e2s)


@jax.jit
def kernel(input_ids, emb0, emb1, emb2):
    m0, m1, m2, cnt = _compute_maps(input_ids)
    counters = cnt[:, :3]
    e0s = jnp.concatenate([emb0[:, :H], emb0[:, H:]], axis=0)
    e1s = jnp.concatenate([emb1[:, :H], emb1[:, H:]], axis=0)
    e2s = jnp.concatenate([emb2[:, :H], emb2[:, H:]], axis=0)
    out = _gather_sum(m0.reshape(ROWS), m1.reshape(ROWS), m2.reshape(ROWS),
                      e0s, e1s, e2s)
    return out.reshape(B, S, D), counters


# speedup vs baseline: 66.0130x; 1.0774x over previous
import functools

import jax
import jax.numpy as jnp
from jax import lax
from jax.experimental import pallas as pl
from jax.experimental.pallas import tpu as pltpu
from jax.experimental.pallas import tpu_sc as plsc

B = 4
S = 2048
D = 1024
H = D // 2
MD0, MD1, MD2 = 2050, 64, 64
OFFSET = 2

NC = 2
NS = 16
ROWS = B * S
ROWS_PER_T = ROWS // NS
CHUNK = 32
NCHUNK = ROWS_PER_T // CHUNK


def _shift_right(x, k, fill):
    pad = jnp.full((B, k), fill, dtype=x.dtype)
    return jnp.concatenate([pad, x[:, : S - k]], axis=1)


def _cumsum(x):
    k = 1
    while k < S:
        x = x + _shift_right(x, k, 0)
        k *= 2
    return x


def _cummax(x, fill):
    k = 1
    while k < S:
        x = jnp.maximum(x, _shift_right(x, k, fill))
        k *= 2
    return x


def _maps_body(ids_ref, m0_ref, m1_ref, m2_ref, cnt_ref):
    tok = ids_ref[...]
    c1 = jnp.logical_and(tok >= 5, tok <= 8)
    c2 = jnp.logical_and(tok >= 9, tok <= 10)
    i32 = jnp.int32
    t = lax.broadcasted_iota(i32, (B, S), 1)
    e = (tok == 1).astype(i32)
    done = _cumsum(e) > 0
    s1 = _cumsum(c1.astype(i32))
    cc2 = _cumsum(c2.astype(i32))
    lastreset = _cummax(jnp.where(jnp.logical_or(c1, c2), t, -1), -1)
    n0raw = jnp.where(lastreset >= 0, t - lastreset, t + 1 + OFFSET)
    ov0 = n0raw == MD0
    n0 = jnp.where(ov0, 0, n0raw)
    v = _cummax(jnp.where(c2, s1, 0), 0)
    n1c = s1 - v
    wrap1 = jnp.logical_and(c1, (n1c & 63) == 0)
    w = _cumsum(wrap1.astype(i32))
    n1 = (n1c & 63) + ov0.astype(i32)
    n2 = (cc2 + w) & 63
    m0_ref[...] = jnp.where(done, MD0 - 1, n0)
    m1_ref[...] = jnp.where(done, MD1 - 1, n1)
    m2_ref[...] = jnp.where(done, MD2 - 1, n2)
    p = jnp.sum(jnp.logical_not(done).astype(i32), axis=1, keepdims=True)
    sel = t == (p - 1)
    f0 = jnp.sum(jnp.where(sel, n0, 0), axis=1, keepdims=True)
    f1 = jnp.sum(jnp.where(sel, n1, 0), axis=1, keepdims=True)
    f2 = jnp.sum(jnp.where(sel, n2, 0), axis=1, keepdims=True)
    f0 = jnp.where(p == 0, OFFSET, f0)
    f1 = jnp.where(p == 0, 0, f1)
    f2 = jnp.where(p == 0, 0, f2)
    col = lax.broadcasted_iota(i32, (B, 128), 1)
    cnt = jnp.where(col == 0, f0, jnp.where(col == 1, f1, jnp.where(col == 2, f2, 0)))
    cnt_ref[...] = cnt


def _compute_maps(input_ids, interpret=False):
    out = pl.pallas_call(
        _maps_body,
        out_shape=[
            jax.ShapeDtypeStruct((B, S), jnp.int32),
            jax.ShapeDtypeStruct((B, S), jnp.int32),
            jax.ShapeDtypeStruct((B, S), jnp.int32),
            jax.ShapeDtypeStruct((B, 128), jnp.int32),
        ],
        interpret=interpret,
    )(input_ids)
    return out


def _gather_body(m0h, m1h, m2h, e0h, e1h, e2h, outh,
                 i0, i1, i2, ba, bb, loc1, loc2, sa, sb, ssa, ssb, si):
    c = lax.axis_index("c")
    s = lax.axis_index("s")
    base = s * ROWS_PER_T

    ci0 = pltpu.async_copy(m0h.at[pl.ds(base, ROWS_PER_T)], i0, si)
    pltpu.sync_copy(e1h.at[pl.ds(c * MD1, MD1)], loc1)
    pltpu.sync_copy(e2h.at[pl.ds(c * MD2, MD2)], loc2)
    ci0.wait()
    ci1 = pltpu.async_copy(m1h.at[pl.ds(base, ROWS_PER_T)], i1, si)
    ci2 = pltpu.async_copy(m2h.at[pl.ds(base, ROWS_PER_T)], i2, si)
    off = c * MD0

    @plsc.parallel_loop(0, ROWS_PER_T // 16, 1, unroll=4)
    def rebase(h):
        sl = pl.ds(h * 16, 16)
        i0.at[sl][...] = i0.at[sl][...] + off

    ci1.wait()
    ci2.wait()

    def do_adds(buf, ci):
        m1s, m2s = [], []
        for h in range(CHUNK // 16):
            sl = pl.ds(ci * CHUNK + h * 16, 16)
            i1v = i1.at[sl][...]
            i2v = i2.at[sl][...]
            for r in range(16):
                m1s.append(i1v[r])
                m2s.append(i2v[r])

        @plsc.parallel_loop(0, H // 16, 1)
        def col_body(cb):
            csl = pl.ds(cb * 16, 16)
            for row in range(CHUNK):
                v = loc1.at[m1s[row], csl][...] + loc2.at[m2s[row], csl][...]
                plsc.addupdate(buf.at[row, csl], v)

    def gather_start(buf, sem, ci):
        return pltpu.async_copy(e0h.at[i0.at[pl.ds(ci * CHUNK, CHUNK)]],
                                buf, sem)

    gather_start(ba, sa, 0)

    def out_ref(ci):
        return outh.at[pl.ds(base + ci * CHUNK, CHUNK), c]

    def step_pair(gi, carry):
        ci = gi * 2

        def one(buf, gsem, ssem, obuf, ogsem, ossem, ci):
            nxt = ci + 1

            @pl.when(jnp.logical_and(nxt < NCHUNK, ci >= 1))
            def _():
                pltpu.make_async_copy(obuf, out_ref(ci - 1), ossem).wait()

            @pl.when(nxt < NCHUNK)
            def _():
                gather_start(obuf, ogsem, nxt)

            pltpu.make_async_copy(e0h.at[i0.at[pl.ds(ci * CHUNK, CHUNK)]],
                                  buf, gsem).wait()
            do_adds(buf, ci)
            pltpu.async_copy(buf, out_ref(ci), ssem)

        one(ba, sa, ssa, bb, sb, ssb, ci)
        one(bb, sb, ssb, ba, sa, ssa, ci + 1)
        return carry

    lax.fori_loop(0, NCHUNK // 2, step_pair, 0, unroll=False)
    pltpu.make_async_copy(ba, out_ref(NCHUNK - 2), ssa).wait()
    pltpu.make_async_copy(bb, out_ref(NCHUNK - 1), ssb).wait()


def _gather_sum(m0f, m1f, m2f, e0s, e1s, e2s):
    mesh = plsc.VectorSubcoreMesh(
        core_axis_name="c", subcore_axis_name="s",
        num_cores=NC, num_subcores=NS)
    kern = pl.kernel(
        _gather_body,
        out_type=jax.ShapeDtypeStruct((ROWS, NC, H), jnp.float32),
        mesh=mesh,
        compiler_params=pltpu.CompilerParams(needs_layout_passes=False),
        scratch_types=[
            pltpu.VMEM((ROWS_PER_T,), jnp.int32),
            pltpu.VMEM((ROWS_PER_T,), jnp.int32),
            pltpu.VMEM((ROWS_PER_T,), jnp.int32),
            pltpu.VMEM((CHUNK, H), jnp.float32),
            pltpu.VMEM((CHUNK, H), jnp.float32),
            pltpu.VMEM((MD1, H), jnp.float32),
            pltpu.VMEM((MD2, H), jnp.float32),
            pltpu.SemaphoreType.DMA,
            pltpu.SemaphoreType.DMA,
            pltpu.SemaphoreType.DMA,
            pltpu.SemaphoreType.DMA,
            pltpu.SemaphoreType.DMA,
        ],
    )
    return kern(m0f, m1f, m2f, e0s, e1s, e2s)


@jax.jit
def kernel(input_ids, emb0, emb1, emb2):
    m0, m1, m2, cnt = _compute_maps(input_ids)
    counters = cnt[:, :3]
    e0s = jnp.concatenate([emb0[:, :H], emb0[:, H:]], axis=0)
    e1s = jnp.concatenate([emb1[:, :H], emb1[:, H:]], axis=0)
    e2s = jnp.concatenate([emb2[:, :H], emb2[:, H:]], axis=0)
    out = _gather_sum(m0.reshape(ROWS), m1.reshape(ROWS), m2.reshape(ROWS),
                      e0s, e1s, e2s)
    return out.reshape(B, S, D), counters
